# baseline (speedup 1.0000x reference)
import functools

import jax
import jax.numpy as jnp
from jax import lax
from jax.experimental import pallas as pl
from jax.experimental.pallas import tpu as pltpu
from jax.experimental.pallas import tpu_sc as plsc

_INPUT = 1024
_HIDDEN = 4096
_BATCH = 64
_SEQ = 128

_CH_ROWS = 256
_CHW = _CH_ROWS * _HIDDEN
_TILE_W = _CHW // 16
_ZW = 16384


def _densify_body(p, ng, cpc, rows_hbm, cols_hbm, vals_hbm, out_hbm,
                  offs_v, tmp_v, vals_v, idx_v, zero_v, chunk_sh):
    c = lax.axis_index("c")
    s = lax.axis_index("s")
    base_e = pl.multiple_of(s * p, 128)
    pltpu.sync_copy(rows_hbm.at[pl.ds(base_e, p)], offs_v)
    pltpu.sync_copy(cols_hbm.at[pl.ds(base_e, p)], tmp_v)
    pltpu.sync_copy(vals_hbm.at[pl.ds(base_e, p)], vals_v)

    def _off(i, carry):
        sl = pl.ds(pl.multiple_of(i * 16, 16), 16)
        offs_v[sl] = tmp_v[sl] * _HIDDEN + offs_v[sl]
        return carry

    lax.fori_loop(0, p // 16, _off, 0)

    def _zero(i, carry):
        zero_v[pl.ds(pl.multiple_of(i * 16, 16), 16)] = jnp.zeros(
            (16,), jnp.float32)
        return carry

    lax.fori_loop(0, _ZW // 16, _zero, 0)

    tile_base = pl.multiple_of(s * _TILE_W, 128)
    for ci in range(cpc):
        chunk_id = c * cpc + ci
        base = chunk_id * _CHW
        for z in range(_TILE_W // _ZW):
            pltpu.sync_copy(
                zero_v,
                chunk_sh.at[pl.ds(tile_base + z * _ZW, _ZW)])
        plsc.subcore_barrier()

        def _group(g, carry):
            gbase = pl.multiple_of(g * 128, 128)
            for j in range(8):
                sl = pl.ds(gbase + j * 16, 16)
                rel = offs_v[sl] - base
                inb = (rel >= 0) & (rel < _CHW)
                idx_v[pl.ds(j * 16, 16)] = jnp.where(inb, rel, _CHW)
            pltpu.sync_copy(vals_v.at[pl.ds(gbase, 128)],
                            chunk_sh.at[idx_v], add=True)
            return carry

        lax.fori_loop(0, ng, _group, 0)
        for j in range(8):
            idx_v[pl.ds(j * 16, 16)] = jnp.full((16,), _CHW, jnp.int32)
        for _ in range(2):
            pltpu.sync_copy(zero_v.at[pl.ds(0, 128)],
                            chunk_sh.at[idx_v], add=True)
        plsc.subcore_barrier()
        pltpu.sync_copy(chunk_sh.at[pl.ds(tile_base, _TILE_W)],
                        out_hbm.at[pl.ds(base + tile_base, _TILE_W)])
        plsc.subcore_barrier()


def _make_densify(n_rows_t, nnz):
    n_chunks = n_rows_t * _HIDDEN // _CHW
    cpc = n_chunks // 2
    ng = -(-nnz // (16 * 128))
    p = ng * 128
    mesh = plsc.VectorSubcoreMesh(core_axis_name="c", subcore_axis_name="s")
    body = functools.partial(_densify_body, p, ng, cpc)
    fn = pl.kernel(
        body,
        out_type=jax.ShapeDtypeStruct((n_rows_t * _HIDDEN,), jnp.float32),
        mesh=mesh,
        scratch_types=[
            pltpu.VMEM((p,), jnp.int32),
            pltpu.VMEM((p,), jnp.int32),
            pltpu.VMEM((p,), jnp.float32),
            pltpu.VMEM((128,), jnp.int32),
            pltpu.VMEM((_ZW,), jnp.float32),
            pltpu.VMEM_SHARED((_CHW + 8,), jnp.float32),
        ],
    )

    def run(rows, cols, vals):
        pad = 16 * p - nnz
        rows = jnp.concatenate([rows, jnp.zeros((pad,), rows.dtype)])
        cols = jnp.concatenate([cols, jnp.zeros((pad,), cols.dtype)])
        vals = jnp.concatenate([vals, jnp.zeros((pad,), vals.dtype)])
        return fn(rows.astype(jnp.int32), cols.astype(jnp.int32), vals)

    return run


_densify_ih = _make_densify(_INPUT, 41943)
_densify_hh = _make_densify(_HIDDEN, 167772)


def _proj_body(x_ref, w_ref, b_ref, u_ref):
    u_ref[...] = jnp.dot(x_ref[...], w_ref[...],
                         preferred_element_type=jnp.float32) + b_ref[...]


def _rnn_body(u_ref, whh_ref, o_ref, h_ref):
    t = pl.program_id(0)

    @pl.when(t == 0)
    def _():
        h_ref[...] = jnp.zeros_like(h_ref)

    pre = u_ref[0] + jnp.dot(h_ref[...].astype(jnp.bfloat16), whh_ref[...],
                             preferred_element_type=jnp.float32)
    h = jnp.tanh(pre)
    h_ref[...] = h
    o_ref[0] = h


_PM = 512


@jax.jit
def _run(xb, w_iht, w_hht, bias_t):
    xall = xb.reshape(_SEQ * _BATCH, _INPUT)
    u = pl.pallas_call(
        _proj_body,
        grid=(_SEQ * _BATCH // _PM,),
        in_specs=[
            pl.BlockSpec((_PM, _INPUT), lambda m: (m, 0)),
            pl.BlockSpec((_INPUT, _HIDDEN), lambda m: (0, 0)),
            pl.BlockSpec((1, _HIDDEN), lambda m: (0, 0)),
        ],
        out_specs=pl.BlockSpec((_PM, _HIDDEN), lambda m: (m, 0)),
        out_shape=jax.ShapeDtypeStruct((_SEQ * _BATCH, _HIDDEN), jnp.float32),
    )(xall, w_iht, bias_t)
    u = u.reshape(_SEQ, _BATCH, _HIDDEN)
    return pl.pallas_call(
        _rnn_body,
        grid=(_SEQ,),
        in_specs=[
            pl.BlockSpec((1, _BATCH, _HIDDEN), lambda t: (t, 0, 0)),
            pl.BlockSpec((_HIDDEN, _HIDDEN), lambda t: (0, 0)),
        ],
        out_specs=pl.BlockSpec((1, _BATCH, _HIDDEN), lambda t: (t, 0, 0)),
        out_shape=jax.ShapeDtypeStruct((_SEQ, _BATCH, _HIDDEN), jnp.float32),
        scratch_shapes=[pltpu.VMEM((_BATCH, _HIDDEN), jnp.float32)],
        compiler_params=pltpu.CompilerParams(
            dimension_semantics=("arbitrary",)),
    )(u, w_hht)


def kernel(x, ih_vals, hh_vals, hh_bias, ih_rows, ih_cols, hh_rows, hh_cols):
    w_iht = _densify_ih(ih_rows, ih_cols, ih_vals).reshape(
        _INPUT, _HIDDEN).astype(jnp.bfloat16)
    w_hht = _densify_hh(hh_rows, hh_cols, hh_vals).reshape(
        _HIDDEN, _HIDDEN).astype(jnp.bfloat16)
    bias_t = jnp.transpose(hh_bias, (1, 0))
    xb = jnp.transpose(x.astype(jnp.bfloat16), (1, 0, 2))
    hs = _run(xb, w_iht, w_hht, bias_t)
    return jnp.transpose(hs, (1, 0, 2))
